# baseline (speedup 1.0000x reference)
import functools

import jax
import jax.numpy as jnp
from jax import lax
from jax.experimental import pallas as pl
from jax.experimental.pallas import tpu as pltpu
from jax.experimental.pallas import tpu_sc as plsc

_B = 16384
_V = 1000000
_D = 64
_H0 = 128
_H1 = 64

_NC = 2
_NS = 16
_NW = _NC * _NS
_B_PER_W = _B // _NW
_NBLK = _V // 128
_TAIL = _NBLK * 128
_TAIL2 = _V - 128


def _sc_gather(emb_t, sidx, spos, tail):
    mesh = plsc.VectorSubcoreMesh(core_axis_name="c", subcore_axis_name="s")

    @functools.partial(
        pl.kernel,
        mesh=mesh,
        compiler_params=pltpu.CompilerParams(needs_layout_passes=False),
        out_type=jax.ShapeDtypeStruct((_B, _D), jnp.float32),
        scratch_types=[
            pltpu.VMEM((_B_PER_W + 16,), jnp.int32),
            pltpu.VMEM((_B_PER_W + 16,), jnp.int32),
            pltpu.VMEM((_D, 128), jnp.float32),
            pltpu.VMEM((_D, 128), jnp.float32),
            pltpu.VMEM((_D, 128), jnp.float32),
            pltpu.VMEM((_D, 128), jnp.float32),
            pltpu.VMEM((_D, 128), jnp.float32),
            pltpu.VMEM((_D, 128), jnp.float32),
            pltpu.VMEM((_D, 128), jnp.float32),
            pltpu.VMEM((_B_PER_W, _D), jnp.float32),
            pltpu.SMEM((_B_PER_W + 1,), jnp.int32),
            pltpu.SMEM((_B_PER_W + 2,), jnp.int32),
            pltpu.SemaphoreType.DMA,
            pltpu.SemaphoreType.DMA,
            pltpu.SemaphoreType.DMA,
            pltpu.SemaphoreType.DMA,
            pltpu.SemaphoreType.DMA,
            pltpu.SemaphoreType.DMA,
            pltpu.SemaphoreType.DMA,
        ],
    )
    def gather_kernel(table_hbm, idx_hbm, pos_hbm, tail_hbm, out_hbm,
                      idx_v, pos_v, buf0, buf1, buf2, buf3, buf4, buf5,
                      tail_v, rows_v, blk_s, start_s,
                      fsem0, fsem1, fsem2, fsem3, fsem4, fsem5, wsem):
        wid = lax.axis_index("s") * _NC + lax.axis_index("c")
        base = wid * _B_PER_W
        pltpu.sync_copy(idx_hbm.at[pl.ds(base, _B_PER_W)],
                        idx_v.at[pl.ds(0, _B_PER_W)])
        pltpu.sync_copy(pos_hbm.at[pl.ds(base, _B_PER_W)],
                        pos_v.at[pl.ds(0, _B_PER_W)])
        pltpu.sync_copy(tail_hbm, tail_v)

        def scan(t, carry):
            prev, n = carry
            blk = idx_v[pl.ds(t, 16)][0] >> 7
            is_new = blk != prev

            @pl.when(is_new)
            def _():
                blk_s[n] = blk
                start_s[n] = t

            return (blk, jnp.where(is_new, n + 1, n))

        _, nblk = lax.fori_loop(0, _B_PER_W, scan, (jnp.int32(-1),
                                                    jnp.int32(0)))
        start_s[nblk] = _B_PER_W

        lanes = lax.iota(jnp.int32, 16)

        def fetch(g, fsem, buf):
            blk = blk_s[g]
            off = pl.multiple_of(jnp.minimum(blk, _NBLK - 1) * 128, 128)
            pltpu.async_copy(table_hbm.at[:, pl.ds(off, 128)], buf, fsem)

        def extract_run(g, buf):
            blk = blk_s[g]
            start = start_s[g]
            end = start_s[g + 1]
            in_tail = blk >= _NBLK
            off = jnp.where(in_tail, _TAIL2, blk * 128)

            def chunk(c, carry):
                t0 = start + c * 16
                cols = idx_v[pl.ds(t0, 16)] - off
                m = lanes < (end - t0)
                trow = t0 + lanes

                @pl.when(jnp.logical_not(in_tail))
                def _():
                    for d in range(_D):
                        vals = plsc.load_gather(buf.at[d], [cols], mask=m)
                        plsc.store_scatter(
                            rows_v, [trow, jnp.full((16,), d, jnp.int32)],
                            vals, mask=m)

                @pl.when(in_tail)
                def _():
                    for d in range(_D):
                        vals = plsc.load_gather(tail_v.at[d], [cols], mask=m)
                        plsc.store_scatter(
                            rows_v, [trow, jnp.full((16,), d, jnp.int32)],
                            vals, mask=m)

                return carry

            lax.fori_loop(0, (end - start + 15) // 16, chunk, 0)

            def write(t, carry):
                p = pos_v[pl.ds(t, 16)][0]
                pltpu.async_copy(
                    rows_v.at[pl.ds(t, 1)], out_hbm.at[pl.ds(p, 1)], wsem
                )
                return carry

            lax.fori_loop(start, end, write, 0)

        bufs = (buf0, buf1, buf2, buf3, buf4, buf5)
        fsems = (fsem0, fsem1, fsem2, fsem3, fsem4, fsem5)
        for r in range(5):
            @pl.when(r < nblk)
            def _(r=r):
                fetch(r, fsems[r], bufs[r])

        def step(g, carry):
            for r in range(6):
                @pl.when(g % 6 == r)
                def _(r=r):
                    @pl.when(g + 5 < nblk)
                    def _(r=r):
                        fetch(g + 5, fsems[(r + 5) % 6], bufs[(r + 5) % 6])
                    pltpu.make_async_copy(
                        table_hbm.at[:, pl.ds(0, 128)], bufs[r], fsems[r]
                    ).wait()
                    extract_run(g, bufs[r])
            return carry

        lax.fori_loop(0, nblk, step, 0)

        def drain(t, carry):
            pltpu.make_async_copy(
                rows_v.at[pl.ds(0, 1)], out_hbm.at[pl.ds(0, 1)], wsem
            ).wait()
            return carry

        lax.fori_loop(0, _B_PER_W, drain, 0)

    return gather_kernel(emb_t, sidx, spos, tail)


_BLK = 4096


def _mlp_body(x_ref, w1_ref, b1_ref, w2t_ref, b2_ref, ot_ref):
    x = x_ref[...]
    h = jnp.dot(x, w1_ref[...], preferred_element_type=jnp.float32)
    h = jnp.maximum(h + b1_ref[...], 0.0)
    ot = lax.dot_general(
        w2t_ref[...], h, (((1,), (1,)), ((), ())),
        preferred_element_type=jnp.float32,
    )
    ot_ref[...] = jnp.maximum(ot + b2_ref[...], 0.0)


def _tc_mlp(x, W1, b1, W2t, b2_col):
    grid = (_B // _BLK,)
    return pl.pallas_call(
        _mlp_body,
        grid=grid,
        in_specs=[
            pl.BlockSpec((_BLK, _D), lambda i: (i, 0)),
            pl.BlockSpec((_D, _H0), lambda i: (0, 0)),
            pl.BlockSpec((1, _H0), lambda i: (0, 0)),
            pl.BlockSpec((_H1, _H0), lambda i: (0, 0)),
            pl.BlockSpec((_H1, 1), lambda i: (0, 0)),
        ],
        out_specs=pl.BlockSpec((_H1, _BLK), lambda i: (0, i)),
        out_shape=jax.ShapeDtypeStruct((_H1, _B), jnp.float32),
    )(x, W1, b1, W2t, b2_col)


def kernel(user_input, emb_table, W1, b1, W2, b2):
    idx = user_input.reshape(-1).astype(jnp.int32)
    pos = lax.iota(jnp.int32, _B)
    sidx, spos = lax.sort_key_val(idx, pos)
    tail = lax.slice(emb_table, (_TAIL2, 0), (_V, _D))
    gathered = _sc_gather(emb_table.T, sidx, spos, tail.T)
    ot = _tc_mlp(gathered, W1, b1.reshape(1, _H0), W2.T, b2.reshape(_H1, 1))
    return ot.T


# speedup vs baseline: 1.2027x; 1.2027x over previous
import functools

import jax
import jax.numpy as jnp
from jax import lax
from jax.experimental import pallas as pl
from jax.experimental.pallas import tpu as pltpu
from jax.experimental.pallas import tpu_sc as plsc

_B = 16384
_V = 1000000
_D = 64
_H0 = 128
_H1 = 64

_NC = 2
_NS = 16
_NW = _NC * _NS
_B_PER_W = _B // _NW
_NBLK = _V // 128
_TAIL = _NBLK * 128
_TAIL2 = _V - 128


def _sc_gather(emb_t, sidx, spos, tail):
    mesh = plsc.VectorSubcoreMesh(core_axis_name="c", subcore_axis_name="s")

    @functools.partial(
        pl.kernel,
        mesh=mesh,
        compiler_params=pltpu.CompilerParams(needs_layout_passes=False),
        out_type=jax.ShapeDtypeStruct((_B, _D), jnp.float32),
        scratch_types=[
            pltpu.VMEM((_B_PER_W + 16,), jnp.int32),
            pltpu.VMEM((_B_PER_W + 16,), jnp.int32),
            pltpu.VMEM((_D, 128), jnp.float32),
            pltpu.VMEM((_D, 128), jnp.float32),
            pltpu.VMEM((_D, 128), jnp.float32),
            pltpu.VMEM((_D, 128), jnp.float32),
            pltpu.VMEM((_D, 128), jnp.float32),
            pltpu.VMEM((_B_PER_W, _D), jnp.float32),
            pltpu.SMEM((_B_PER_W + 1,), jnp.int32),
            pltpu.SMEM((_B_PER_W + 2,), jnp.int32),
            pltpu.SemaphoreType.DMA,
            pltpu.SemaphoreType.DMA,
            pltpu.SemaphoreType.DMA,
            pltpu.SemaphoreType.DMA,
            pltpu.SemaphoreType.DMA,
        ],
    )
    def gather_kernel(table_hbm, idx_hbm, pos_hbm, tail_hbm, out_hbm,
                      idx_v, pos_v, buf0, buf1, buf2, buf3, tail_v, rows_v,
                      blk_s, start_s, fsem0, fsem1, fsem2, fsem3, wsem):
        wid = lax.axis_index("s") * _NC + lax.axis_index("c")
        base = wid * _B_PER_W
        pltpu.sync_copy(idx_hbm.at[pl.ds(base, _B_PER_W)],
                        idx_v.at[pl.ds(0, _B_PER_W)])
        pltpu.sync_copy(pos_hbm.at[pl.ds(base, _B_PER_W)],
                        pos_v.at[pl.ds(0, _B_PER_W)])
        pltpu.sync_copy(tail_hbm, tail_v)

        def scan(t, carry):
            prev, n = carry
            blk = idx_v[pl.ds(t, 16)][0] >> 7
            is_new = blk != prev

            @pl.when(is_new)
            def _():
                blk_s[n] = blk
                start_s[n] = t

            return (blk, jnp.where(is_new, n + 1, n))

        _, nblk = lax.fori_loop(0, _B_PER_W, scan, (jnp.int32(-1),
                                                    jnp.int32(0)))
        start_s[nblk] = _B_PER_W

        lanes = lax.iota(jnp.int32, 16)

        def fetch(g, fsem, buf):
            blk = blk_s[g]
            off = pl.multiple_of(jnp.minimum(blk, _NBLK - 1) * 128, 128)
            pltpu.async_copy(
                table_hbm.at[pl.ds(0, 32), pl.ds(off, 128)],
                buf.at[pl.ds(0, 32)], fsem)
            pltpu.async_copy(
                table_hbm.at[pl.ds(32, 32), pl.ds(off, 128)],
                buf.at[pl.ds(32, 32)], fsem)

        def extract_run(g, buf):
            blk = blk_s[g]
            start = start_s[g]
            end = start_s[g + 1]
            in_tail = blk >= _NBLK
            off = jnp.where(in_tail, _TAIL2, blk * 128)

            def chunk(c, carry):
                t0 = start + c * 16
                cols = idx_v[pl.ds(t0, 16)] - off
                m = lanes < (end - t0)
                trow = t0 + lanes

                @pl.when(jnp.logical_not(in_tail))
                def _():
                    for d in range(_D):
                        vals = plsc.load_gather(buf.at[d], [cols], mask=m)
                        plsc.store_scatter(
                            rows_v, [trow, jnp.full((16,), d, jnp.int32)],
                            vals, mask=m)

                @pl.when(in_tail)
                def _():
                    for d in range(_D):
                        vals = plsc.load_gather(tail_v.at[d], [cols], mask=m)
                        plsc.store_scatter(
                            rows_v, [trow, jnp.full((16,), d, jnp.int32)],
                            vals, mask=m)

                return carry

            lax.fori_loop(0, (end - start + 15) // 16, chunk, 0)

            def write(t, carry):
                p = pos_v[pl.ds(t, 16)][0]
                pltpu.async_copy(
                    rows_v.at[pl.ds(t, 1)], out_hbm.at[pl.ds(p, 1)], wsem
                )
                return carry

            lax.fori_loop(start, end, write, 0)

        bufs = (buf0, buf1, buf2, buf3)
        fsems = (fsem0, fsem1, fsem2, fsem3)
        for r in range(3):
            @pl.when(r < nblk)
            def _(r=r):
                fetch(r, fsems[r], bufs[r])

        def step(g, carry):
            for r in range(4):
                @pl.when(g % 4 == r)
                def _(r=r):
                    @pl.when(g + 3 < nblk)
                    def _(r=r):
                        fetch(g + 3, fsems[(r + 3) % 4], bufs[(r + 3) % 4])
                    pltpu.make_async_copy(
                        table_hbm.at[:, pl.ds(0, 128)], bufs[r], fsems[r]
                    ).wait()
                    extract_run(g, bufs[r])
            return carry

        lax.fori_loop(0, nblk, step, 0)

        def drain(t, carry):
            pltpu.make_async_copy(
                rows_v.at[pl.ds(0, 1)], out_hbm.at[pl.ds(0, 1)], wsem
            ).wait()
            return carry

        lax.fori_loop(0, _B_PER_W, drain, 0)

    return gather_kernel(emb_t, sidx, spos, tail)


_BLK = 2048


def _mlp_body(x_ref, w1_ref, b1_ref, w2t_ref, b2_ref, ot_ref):
    x = x_ref[...]
    h = jnp.dot(x, w1_ref[...], preferred_element_type=jnp.float32)
    h = jnp.maximum(h + b1_ref[...], 0.0)
    ot = lax.dot_general(
        w2t_ref[...], h, (((1,), (1,)), ((), ())),
        preferred_element_type=jnp.float32,
    )
    ot_ref[...] = jnp.maximum(ot + b2_ref[...], 0.0)


def _tc_mlp(x, W1, b1, W2t, b2_col):
    grid = (_B // _BLK,)
    return pl.pallas_call(
        _mlp_body,
        grid=grid,
        in_specs=[
            pl.BlockSpec((_BLK, _D), lambda i: (i, 0)),
            pl.BlockSpec((_D, _H0), lambda i: (0, 0)),
            pl.BlockSpec((1, _H0), lambda i: (0, 0)),
            pl.BlockSpec((_H1, _H0), lambda i: (0, 0)),
            pl.BlockSpec((_H1, 1), lambda i: (0, 0)),
        ],
        out_specs=pl.BlockSpec((_H1, _BLK), lambda i: (0, i)),
        out_shape=jax.ShapeDtypeStruct((_H1, _B), jnp.float32),
    )(x, W1, b1, W2t, b2_col)


def kernel(user_input, emb_table, W1, b1, W2, b2):
    idx = user_input.reshape(-1).astype(jnp.int32)
    pos = lax.iota(jnp.int32, _B)
    sidx, spos = lax.sort_key_val(idx, pos)
    tail = lax.slice(emb_table, (_TAIL2, 0), (_V, _D))
    gathered = _sc_gather(emb_table.T, sidx, spos, tail.T)
    ot = _tc_mlp(gathered, W1, b1.reshape(1, _H0), W2.T, b2.reshape(_H1, 1))
    return ot.T
